# baseline (speedup 1.0000x reference)
import jax
import jax.numpy as jnp
from jax.experimental import pallas as pl
from jax.experimental.pallas import tpu as pltpu

E = 64
TOPK = 2
S_TILE = 1024
N_S = 2


def _router_kernel(x_ref, wr_ref, br_ref, idx_ref, w_ref):
    xm = jnp.mean(x_ref[0], axis=0, keepdims=True)
    logits = jnp.dot(xm, wr_ref[...], preferred_element_type=jnp.float32)
    logits = logits + br_ref[...][None, :]
    m = jnp.max(logits, axis=-1, keepdims=True)
    p = jnp.exp(logits - m)
    p = p / jnp.sum(p, axis=-1, keepdims=True)
    ids = jax.lax.broadcasted_iota(jnp.int32, p.shape, 1)
    big = jnp.int32(E)
    m1 = jnp.max(p, axis=-1, keepdims=True)
    i1 = jnp.min(jnp.where(p == m1, ids, big), axis=-1, keepdims=True)
    p2 = jnp.where(ids == i1, -1.0, p)
    m2 = jnp.max(p2, axis=-1, keepdims=True)
    i2 = jnp.min(jnp.where(p2 == m2, ids, big), axis=-1, keepdims=True)
    t = jnp.exp(m2 - m1)
    w1 = 1.0 / (1.0 + t)
    w2 = t / (1.0 + t)
    b = pl.program_id(0)
    lane = jax.lax.broadcasted_iota(jnp.int32, idx_ref.shape, 1)
    cur_i = jnp.where(lane == b * TOPK, i1, idx_ref[...])
    idx_ref[...] = jnp.where(lane == b * TOPK + 1, i2, cur_i).astype(jnp.int32)
    cur_w = jnp.where(lane == b * TOPK, w1, w_ref[...])
    w_ref[...] = jnp.where(lane == b * TOPK + 1, w2, cur_w)


def _weight_copies(w1_hbm, w2_hbm, idx_sref, batch, side,
                   w1a_buf, w1b_buf, w2a_buf, w2b_buf, sems):
    ia = idx_sref[0, batch * TOPK]
    ib = idx_sref[0, batch * TOPK + 1]
    return (
        pltpu.make_async_copy(w1_hbm.at[ia], w1a_buf.at[side], sems.at[side, 0]),
        pltpu.make_async_copy(w2_hbm.at[ia], w2a_buf.at[side], sems.at[side, 1]),
        pltpu.make_async_copy(w1_hbm.at[ib], w1b_buf.at[side], sems.at[side, 2]),
        pltpu.make_async_copy(w2_hbm.at[ib], w2b_buf.at[side], sems.at[side, 3]),
    )


def _moe_kernel(idx_sref, w_sref, x_ref, b1_ref, b2_ref, gamma_ref, beta_ref,
                w1_hbm, w2_hbm, out_ref,
                w1a_buf, w1b_buf, w2a_buf, w2b_buf, sems):
    b = pl.program_id(0)
    s = pl.program_id(1)
    nb = pl.num_programs(0)
    par = jax.lax.rem(b, 2)

    @pl.when(jnp.logical_and(b == 0, s == 0))
    def _issue_first():
        for cp in _weight_copies(w1_hbm, w2_hbm, idx_sref, 0, 0,
                                 w1a_buf, w1b_buf, w2a_buf, w2b_buf, sems):
            cp.start()

    @pl.when(jnp.logical_and(s == 0, b + 1 < nb))
    def _issue_next():
        for cp in _weight_copies(w1_hbm, w2_hbm, idx_sref, b + 1, 1 - par,
                                 w1a_buf, w1b_buf, w2a_buf, w2b_buf, sems):
            cp.start()

    cps = _weight_copies(w1_hbm, w2_hbm, idx_sref, b, par,
                         w1a_buf, w1b_buf, w2a_buf, w2b_buf, sems)

    @pl.when(s == 0)
    def _wait_a():
        cps[0].wait()
        cps[1].wait()

    ia = idx_sref[0, b * TOPK]
    ib = idx_sref[0, b * TOPK + 1]
    wa = w_sref[0, b * TOPK]
    wb = w_sref[0, b * TOPK + 1]
    xb = x_ref[0]
    b1a = b1_ref[pl.ds(ia, 1), :]
    b1b = b1_ref[pl.ds(ib, 1), :]
    b2a = b2_ref[pl.ds(ia, 1), :]
    b2b = b2_ref[pl.ds(ib, 1), :]

    ha = jnp.maximum(
        jnp.dot(xb, w1a_buf[par], preferred_element_type=jnp.float32) + b1a,
        0.0)
    oa = jnp.dot(ha, w2a_buf[par], preferred_element_type=jnp.float32)

    @pl.when(s == 0)
    def _wait_b():
        cps[2].wait()
        cps[3].wait()

    hb = jnp.maximum(
        jnp.dot(xb, w1b_buf[par], preferred_element_type=jnp.float32) + b1b,
        0.0)
    ob = jnp.dot(hb, w2b_buf[par], preferred_element_type=jnp.float32)

    y = (oa + b2a) * wa + (ob + b2b) * wb + xb
    mu = jnp.mean(y, axis=-1, keepdims=True)
    yc = y - mu
    var = jnp.mean(yc * yc, axis=-1, keepdims=True)
    out_ref[0] = yc * jax.lax.rsqrt(var + 1e-5) * gamma_ref[...] + beta_ref[...]


@jax.jit
def kernel(x, Wr, br, W1, b1, W2, b2, gamma, beta):
    B, S, D = x.shape
    F = W1.shape[2]

    idx, w = pl.pallas_call(
        _router_kernel,
        grid=(B,),
        in_specs=[
            pl.BlockSpec((1, S, D), lambda b: (b, 0, 0)),
            pl.BlockSpec((D, E), lambda b: (0, 0)),
            pl.BlockSpec((E,), lambda b: (0,)),
        ],
        out_specs=[
            pl.BlockSpec((1, B * TOPK), lambda b: (0, 0)),
            pl.BlockSpec((1, B * TOPK), lambda b: (0, 0)),
        ],
        out_shape=[
            jax.ShapeDtypeStruct((1, B * TOPK), jnp.int32),
            jax.ShapeDtypeStruct((1, B * TOPK), jnp.float32),
        ],
    )(x, Wr, br)

    idx_flat = idx
    w_flat = w

    out = pl.pallas_call(
        _moe_kernel,
        grid_spec=pltpu.PrefetchScalarGridSpec(
            num_scalar_prefetch=2,
            grid=(B, N_S),
            in_specs=[
                pl.BlockSpec((1, S_TILE, D), lambda b, s, idx, w: (b, s, 0)),
                pl.BlockSpec((E, F), lambda b, s, idx, w: (0, 0)),
                pl.BlockSpec((E, D), lambda b, s, idx, w: (0, 0)),
                pl.BlockSpec((D,), lambda b, s, idx, w: (0,)),
                pl.BlockSpec((D,), lambda b, s, idx, w: (0,)),
                pl.BlockSpec(memory_space=pltpu.MemorySpace.HBM),
                pl.BlockSpec(memory_space=pltpu.MemorySpace.HBM),
            ],
            out_specs=pl.BlockSpec((1, S_TILE, D), lambda b, s, idx, w: (b, s, 0)),
            scratch_shapes=[
                pltpu.VMEM((2, D, F), jnp.float32),
                pltpu.VMEM((2, D, F), jnp.float32),
                pltpu.VMEM((2, F, D), jnp.float32),
                pltpu.VMEM((2, F, D), jnp.float32),
                pltpu.SemaphoreType.DMA((2, 4)),
            ],
        ),
        out_shape=jax.ShapeDtypeStruct((B, S, D), jnp.float32),
    )(idx_flat, w_flat, x, b1, b2, gamma, beta, W1, W2)

    return out


# speedup vs baseline: 1.0946x; 1.0946x over previous
import jax
import jax.numpy as jnp
from jax.experimental import pallas as pl
from jax.experimental.pallas import tpu as pltpu

E = 64
TOPK = 2
S_TILE = 1024
N_S = 2


def _router_kernel(x_ref, wr_ref, br_ref, idx_ref, w_ref):
    xm = jnp.mean(x_ref[0], axis=0, keepdims=True)
    logits = jnp.dot(xm, wr_ref[...], preferred_element_type=jnp.float32)
    logits = logits + br_ref[...][None, :]
    m = jnp.max(logits, axis=-1, keepdims=True)
    p = jnp.exp(logits - m)
    p = p / jnp.sum(p, axis=-1, keepdims=True)
    ids = jax.lax.broadcasted_iota(jnp.int32, p.shape, 1)
    big = jnp.int32(E)
    m1 = jnp.max(p, axis=-1, keepdims=True)
    i1 = jnp.min(jnp.where(p == m1, ids, big), axis=-1, keepdims=True)
    p2 = jnp.where(ids == i1, -1.0, p)
    m2 = jnp.max(p2, axis=-1, keepdims=True)
    i2 = jnp.min(jnp.where(p2 == m2, ids, big), axis=-1, keepdims=True)
    t = jnp.exp(m2 - m1)
    w1 = 1.0 / (1.0 + t)
    w2 = t / (1.0 + t)
    b = pl.program_id(0)
    lane = jax.lax.broadcasted_iota(jnp.int32, idx_ref.shape, 1)
    cur_i = jnp.where(lane == b * TOPK, i1, idx_ref[...])
    idx_ref[...] = jnp.where(lane == b * TOPK + 1, i2, cur_i).astype(jnp.int32)
    cur_w = jnp.where(lane == b * TOPK, w1, w_ref[...])
    w_ref[...] = jnp.where(lane == b * TOPK + 1, w2, cur_w)


def _weight_copies(w1_hbm, w2_hbm, idx_sref, batch, side,
                   w1a_buf, w1b_buf, w2a_buf, w2b_buf, sems):
    ia = idx_sref[0, batch * TOPK]
    ib = idx_sref[0, batch * TOPK + 1]
    return (
        pltpu.make_async_copy(w1_hbm.at[ia], w1a_buf.at[side], sems.at[side, 0]),
        pltpu.make_async_copy(w2_hbm.at[ia], w2a_buf.at[side], sems.at[side, 1]),
        pltpu.make_async_copy(w1_hbm.at[ib], w1b_buf.at[side], sems.at[side, 2]),
        pltpu.make_async_copy(w2_hbm.at[ib], w2b_buf.at[side], sems.at[side, 3]),
    )


def _moe_kernel(idx_sref, w_sref, x_ref, b1_ref, b2_ref, gamma_ref, beta_ref,
                w1_hbm, w2_hbm, out_ref,
                w1a_buf, w1b_buf, w2a_buf, w2b_buf, sems):
    b = pl.program_id(0)
    s = pl.program_id(1)
    nb = pl.num_programs(0)
    par = jax.lax.rem(b, 2)

    @pl.when(jnp.logical_and(b == 0, s == 0))
    def _issue_first():
        for cp in _weight_copies(w1_hbm, w2_hbm, idx_sref, 0, 0,
                                 w1a_buf, w1b_buf, w2a_buf, w2b_buf, sems):
            cp.start()

    @pl.when(jnp.logical_and(s == 0, b + 1 < nb))
    def _issue_next():
        for cp in _weight_copies(w1_hbm, w2_hbm, idx_sref, b + 1, 1 - par,
                                 w1a_buf, w1b_buf, w2a_buf, w2b_buf, sems):
            cp.start()

    @pl.when(s == 0)
    def _wait_current():
        for cp in _weight_copies(w1_hbm, w2_hbm, idx_sref, b, par,
                                 w1a_buf, w1b_buf, w2a_buf, w2b_buf, sems):
            cp.wait()

    ia = idx_sref[0, b * TOPK]
    ib = idx_sref[0, b * TOPK + 1]
    wa = w_sref[0, b * TOPK]
    wb = w_sref[0, b * TOPK + 1]
    xb = x_ref[0]
    b1a = b1_ref[pl.ds(ia, 1), :]
    b1b = b1_ref[pl.ds(ib, 1), :]
    b2a = b2_ref[pl.ds(ia, 1), :]
    b2b = b2_ref[pl.ds(ib, 1), :]

    ha = jnp.maximum(
        jnp.dot(xb, w1a_buf[par], preferred_element_type=jnp.float32) + b1a,
        0.0)
    oa = jnp.dot(ha, w2a_buf[par], preferred_element_type=jnp.float32)
    hb = jnp.maximum(
        jnp.dot(xb, w1b_buf[par], preferred_element_type=jnp.float32) + b1b,
        0.0)
    ob = jnp.dot(hb, w2b_buf[par], preferred_element_type=jnp.float32)

    y = (oa + b2a) * wa + (ob + b2b) * wb + xb
    mu = jnp.mean(y, axis=-1, keepdims=True)
    yc = y - mu
    var = jnp.mean(yc * yc, axis=-1, keepdims=True)
    out_ref[0] = yc * jax.lax.rsqrt(var + 1e-5) * gamma_ref[...] + beta_ref[...]


@jax.jit
def kernel(x, Wr, br, W1, b1, W2, b2, gamma, beta):
    B, S, D = x.shape
    F = W1.shape[2]

    idx, w = pl.pallas_call(
        _router_kernel,
        grid=(B,),
        in_specs=[
            pl.BlockSpec((1, S, D), lambda b: (b, 0, 0)),
            pl.BlockSpec((D, E), lambda b: (0, 0)),
            pl.BlockSpec((E,), lambda b: (0,)),
        ],
        out_specs=[
            pl.BlockSpec((1, B * TOPK), lambda b: (0, 0)),
            pl.BlockSpec((1, B * TOPK), lambda b: (0, 0)),
        ],
        out_shape=[
            jax.ShapeDtypeStruct((1, B * TOPK), jnp.int32),
            jax.ShapeDtypeStruct((1, B * TOPK), jnp.float32),
        ],
    )(x, Wr, br)

    idx_flat = idx
    w_flat = w

    out = pl.pallas_call(
        _moe_kernel,
        grid_spec=pltpu.PrefetchScalarGridSpec(
            num_scalar_prefetch=2,
            grid=(B, N_S),
            in_specs=[
                pl.BlockSpec((1, S_TILE, D), lambda b, s, idx, w: (b, s, 0)),
                pl.BlockSpec((E, F), lambda b, s, idx, w: (0, 0)),
                pl.BlockSpec((E, D), lambda b, s, idx, w: (0, 0)),
                pl.BlockSpec((D,), lambda b, s, idx, w: (0,)),
                pl.BlockSpec((D,), lambda b, s, idx, w: (0,)),
                pl.BlockSpec(memory_space=pltpu.MemorySpace.HBM),
                pl.BlockSpec(memory_space=pltpu.MemorySpace.HBM),
            ],
            out_specs=pl.BlockSpec((1, S_TILE, D), lambda b, s, idx, w: (b, s, 0)),
            scratch_shapes=[
                pltpu.VMEM((2, D, F), jnp.float32),
                pltpu.VMEM((2, D, F), jnp.float32),
                pltpu.VMEM((2, F, D), jnp.float32),
                pltpu.VMEM((2, F, D), jnp.float32),
                pltpu.SemaphoreType.DMA((2, 4)),
            ],
        ),
        out_shape=jax.ShapeDtypeStruct((B, S, D), jnp.float32),
    )(idx_flat, w_flat, x, b1, b2, gamma, beta, W1, W2)

    return out


# speedup vs baseline: 1.0976x; 1.0027x over previous
import jax
import jax.numpy as jnp
from jax.experimental import pallas as pl
from jax.experimental.pallas import tpu as pltpu

E = 64
TOPK = 2
S_TILE = 1024
N_S = 2


def _router_kernel(x_ref, wr_ref, br_ref, idx_ref, w_ref):
    xm = jnp.mean(x_ref[0], axis=0, keepdims=True)
    logits = jnp.dot(xm, wr_ref[...], preferred_element_type=jnp.float32)
    logits = logits + br_ref[...][None, :]
    m = jnp.max(logits, axis=-1, keepdims=True)
    p = jnp.exp(logits - m)
    p = p / jnp.sum(p, axis=-1, keepdims=True)
    ids = jax.lax.broadcasted_iota(jnp.int32, p.shape, 1)
    big = jnp.int32(E)
    m1 = jnp.max(p, axis=-1, keepdims=True)
    i1 = jnp.min(jnp.where(p == m1, ids, big), axis=-1, keepdims=True)
    p2 = jnp.where(ids == i1, -1.0, p)
    m2 = jnp.max(p2, axis=-1, keepdims=True)
    i2 = jnp.min(jnp.where(p2 == m2, ids, big), axis=-1, keepdims=True)
    t = jnp.exp(m2 - m1)
    w1 = 1.0 / (1.0 + t)
    w2 = t / (1.0 + t)
    b = pl.program_id(0)
    lane = jax.lax.broadcasted_iota(jnp.int32, idx_ref.shape, 1)
    cur_i = jnp.where(lane == b * TOPK, i1, idx_ref[...])
    idx_ref[...] = jnp.where(lane == b * TOPK + 1, i2, cur_i).astype(jnp.int32)
    cur_w = jnp.where(lane == b * TOPK, w1, w_ref[...])
    w_ref[...] = jnp.where(lane == b * TOPK + 1, w2, cur_w)


def _weight_copies(w1_hbm, w2_hbm, idx_sref, batch, side,
                   w1a_buf, w1b_buf, w2a_buf, w2b_buf, sems):
    ia = idx_sref[0, batch * TOPK]
    ib = idx_sref[0, batch * TOPK + 1]
    return (
        pltpu.make_async_copy(w1_hbm.at[ia], w1a_buf.at[side], sems.at[side, 0]),
        pltpu.make_async_copy(w2_hbm.at[ia], w2a_buf.at[side], sems.at[side, 1]),
        pltpu.make_async_copy(w1_hbm.at[ib], w1b_buf.at[side], sems.at[side, 2]),
        pltpu.make_async_copy(w2_hbm.at[ib], w2b_buf.at[side], sems.at[side, 3]),
    )


def _moe_kernel(idx_sref, w_sref, x_ref, b1_ref, b2_ref, gamma_ref, beta_ref,
                w1_hbm, w2_hbm, out_ref,
                w1a_buf, w1b_buf, w2a_buf, w2b_buf, sems):
    b = pl.program_id(0)
    s = pl.program_id(1)
    nb = pl.num_programs(0)
    par = jax.lax.rem(b, 2)

    @pl.when(jnp.logical_and(b == 0, s == 0))
    def _issue_first():
        for cp in _weight_copies(w1_hbm, w2_hbm, idx_sref, 0, 0,
                                 w1a_buf, w1b_buf, w2a_buf, w2b_buf, sems):
            cp.start()

    @pl.when(s == 0)
    def _wait_current():
        for cp in _weight_copies(w1_hbm, w2_hbm, idx_sref, b, par,
                                 w1a_buf, w1b_buf, w2a_buf, w2b_buf, sems):
            cp.wait()

    ia = idx_sref[0, b * TOPK]
    ib = idx_sref[0, b * TOPK + 1]
    wa = w_sref[0, b * TOPK]
    wb = w_sref[0, b * TOPK + 1]
    xb = x_ref[0]
    b1a = b1_ref[pl.ds(ia, 1), :]
    b1b = b1_ref[pl.ds(ib, 1), :]
    b2a = b2_ref[pl.ds(ia, 1), :]
    b2b = b2_ref[pl.ds(ib, 1), :]

    ha = jnp.maximum(
        jnp.dot(xb, w1a_buf[par], preferred_element_type=jnp.float32) + b1a,
        0.0)
    oa = jnp.dot(ha, w2a_buf[par], preferred_element_type=jnp.float32)
    hb = jnp.maximum(
        jnp.dot(xb, w1b_buf[par], preferred_element_type=jnp.float32) + b1b,
        0.0)
    ob = jnp.dot(hb, w2b_buf[par], preferred_element_type=jnp.float32)

    @pl.when(jnp.logical_and(s == 0, b + 1 < nb))
    def _issue_next():
        for cp in _weight_copies(w1_hbm, w2_hbm, idx_sref, b + 1, 1 - par,
                                 w1a_buf, w1b_buf, w2a_buf, w2b_buf, sems):
            cp.start()

    y = (oa + b2a) * wa + (ob + b2b) * wb + xb
    mu = jnp.mean(y, axis=-1, keepdims=True)
    yc = y - mu
    var = jnp.mean(yc * yc, axis=-1, keepdims=True)
    out_ref[0] = yc * jax.lax.rsqrt(var + 1e-5) * gamma_ref[...] + beta_ref[...]


@jax.jit
def kernel(x, Wr, br, W1, b1, W2, b2, gamma, beta):
    B, S, D = x.shape
    F = W1.shape[2]

    idx, w = pl.pallas_call(
        _router_kernel,
        grid=(B,),
        in_specs=[
            pl.BlockSpec((1, S, D), lambda b: (b, 0, 0)),
            pl.BlockSpec((D, E), lambda b: (0, 0)),
            pl.BlockSpec((E,), lambda b: (0,)),
        ],
        out_specs=[
            pl.BlockSpec((1, B * TOPK), lambda b: (0, 0)),
            pl.BlockSpec((1, B * TOPK), lambda b: (0, 0)),
        ],
        out_shape=[
            jax.ShapeDtypeStruct((1, B * TOPK), jnp.int32),
            jax.ShapeDtypeStruct((1, B * TOPK), jnp.float32),
        ],
    )(x, Wr, br)

    idx_flat = idx
    w_flat = w

    out = pl.pallas_call(
        _moe_kernel,
        grid_spec=pltpu.PrefetchScalarGridSpec(
            num_scalar_prefetch=2,
            grid=(B, N_S),
            in_specs=[
                pl.BlockSpec((1, S_TILE, D), lambda b, s, idx, w: (b, s, 0)),
                pl.BlockSpec((E, F), lambda b, s, idx, w: (0, 0)),
                pl.BlockSpec((E, D), lambda b, s, idx, w: (0, 0)),
                pl.BlockSpec((D,), lambda b, s, idx, w: (0,)),
                pl.BlockSpec((D,), lambda b, s, idx, w: (0,)),
                pl.BlockSpec(memory_space=pltpu.MemorySpace.HBM),
                pl.BlockSpec(memory_space=pltpu.MemorySpace.HBM),
            ],
            out_specs=pl.BlockSpec((1, S_TILE, D), lambda b, s, idx, w: (b, s, 0)),
            scratch_shapes=[
                pltpu.VMEM((2, D, F), jnp.float32),
                pltpu.VMEM((2, D, F), jnp.float32),
                pltpu.VMEM((2, F, D), jnp.float32),
                pltpu.VMEM((2, F, D), jnp.float32),
                pltpu.SemaphoreType.DMA((2, 4)),
            ],
        ),
        out_shape=jax.ShapeDtypeStruct((B, S, D), jnp.float32),
    )(idx_flat, w_flat, x, b1, b2, gamma, beta, W1, W2)

    return out
